# Initial kernel scaffold; baseline (speedup 1.0000x reference)
import jax
import jax.numpy as jnp
from jax.experimental import pallas as pl


def kernel(x, emb1, emb2, emb3, W1, b1, W2, b2):
    raise NotImplementedError("write your pallas kernel here")


# speedup vs baseline: 3.3095x; 3.3095x over previous
import functools

import jax
import jax.numpy as jnp
from jax import lax
from jax.experimental import pallas as pl
from jax.experimental.pallas import tpu as pltpu
from jax.experimental.pallas import tpu_sc as plsc

VOCAB = 100000
D = 128
H = 32
C = 1000
B = 4096
L = 50

NC = 2
NS = 16
LANES = 16
NW = NC * NS
SPW = B // NW
PAIRS = SPW // 2
ROWS = 2 * L
IDX_PAD = 128
BT = 256


def _pool_body(e1_hbm, e2_hbm, e3_hbm, idx_hbm, out_hbm,
               idx_v, rows_v, acc_v, sem):
    wid = lax.axis_index("s") * NC + lax.axis_index("c")
    for t, emb in enumerate((e1_hbm, e2_hbm, e3_hbm)):
        pltpu.sync_copy(idx_hbm.at[t, wid], idx_v)

        @pl.loop(0, SPW)
        def _(sm):
            for c in range(D // LANES):
                acc_v[sm, pl.ds(c * LANES, LANES)] = jnp.zeros((LANES,),
                                                               jnp.float32)

        @pl.loop(0, PAIRS)
        def _(j):
            pltpu.async_copy(emb.at[idx_v.at[j, pl.ds(0, ROWS)]],
                             rows_v, sem).wait()
            for s in range(2):
                @pl.loop(0, L)
                def _(l, s=s):
                    r = s * L + l
                    for c in range(D // LANES):
                        sl = pl.ds(c * LANES, LANES)
                        plsc.addupdate(acc_v.at[2 * j + s, sl], rows_v[r, sl])

        pltpu.sync_copy(acc_v, out_hbm.at[t].at[pl.ds(wid * SPW, SPW)])


def _mlp_body(p_ref, w1_ref, b1_ref, w2_ref, b2_ref, o_ref):
    p = p_ref[...]
    w1 = w1_ref[...]
    h = (jnp.dot(p[0], w1[0], preferred_element_type=jnp.float32)
         + jnp.dot(p[1], w1[1], preferred_element_type=jnp.float32)
         + jnp.dot(p[2], w1[2], preferred_element_type=jnp.float32)
         + b1_ref[...])
    h = jnp.maximum(h, 0.0)
    o_ref[...] = (jnp.dot(h, w2_ref[...], preferred_element_type=jnp.float32)
                  + b2_ref[...])


def kernel(x, emb1, emb2, emb3, W1, b1, W2, b2):
    x = x.astype(jnp.int32)
    idx = x.reshape(3, B // 2, ROWS)
    idx = jnp.pad(idx, ((0, 0), (0, 0), (0, IDX_PAD - ROWS)))
    idx = idx.reshape(3, NW, PAIRS, IDX_PAD)

    mesh = plsc.VectorSubcoreMesh(core_axis_name="c", subcore_axis_name="s")
    pooled = pl.kernel(
        _pool_body,
        out_type=jax.ShapeDtypeStruct((3, B, D), jnp.float32),
        mesh=mesh,
        scratch_types=[
            pltpu.VMEM((PAIRS, IDX_PAD), jnp.int32),
            pltpu.VMEM((ROWS, D), jnp.float32),
            pltpu.VMEM((SPW, D), jnp.float32),
            pltpu.SemaphoreType.DMA,
        ],
    )(emb1, emb2, emb3, idx)

    w1s = (W1 * (1.0 / L)).reshape(3, D, H)
    b1r = b1.reshape(1, H)
    b2r = b2.reshape(1, C)

    out = pl.pallas_call(
        _mlp_body,
        grid=(B // BT,),
        in_specs=[
            pl.BlockSpec((3, BT, D), lambda i: (0, i, 0)),
            pl.BlockSpec((3, D, H), lambda i: (0, 0, 0)),
            pl.BlockSpec((1, H), lambda i: (0, 0)),
            pl.BlockSpec((H, C), lambda i: (0, 0)),
            pl.BlockSpec((1, C), lambda i: (0, 0)),
        ],
        out_specs=pl.BlockSpec((BT, C), lambda i: (i, 0)),
        out_shape=jax.ShapeDtypeStruct((B, C), jnp.float32),
    )(pooled, w1s, b1r, W2, b2r)
    return out


# speedup vs baseline: 11.5932x; 3.5030x over previous
import functools

import jax
import jax.numpy as jnp
from jax import lax
from jax.experimental import pallas as pl
from jax.experimental.pallas import tpu as pltpu
from jax.experimental.pallas import tpu_sc as plsc

VOCAB = 100000
D = 128
H = 32
C = 1000
B = 4096
L = 50

NC = 2
NS = 16
LANES = 16
NW = NC * NS
SPW = B // NW
PAIRS = SPW // 2
ROWS = 2 * L
IDX_PAD = 128
BT = 256


def _pool_body(e1_hbm, e2_hbm, e3_hbm, idx_hbm, out_hbm,
               idx_v, rows0, rows1, acc_v, sem0, sem1):
    wid = lax.axis_index("s") * NC + lax.axis_index("c")
    nch = D // LANES
    bufs = ((rows0, sem0), (rows1, sem1))

    for t, emb in enumerate((e1_hbm, e2_hbm, e3_hbm)):
        pltpu.sync_copy(idx_hbm.at[t, wid], idx_v)

        for b, (buf, sem) in enumerate(bufs):
            pltpu.async_copy(emb.at[idx_v.at[b, pl.ds(0, ROWS)]], buf, sem)

        @pl.loop(0, PAIRS, step=2)
        def _(j, emb=emb):
            for b, (buf, sem) in enumerate(bufs):
                jj = j + b
                pltpu.make_async_copy(
                    emb.at[idx_v.at[0, pl.ds(0, ROWS)]], buf, sem).wait()
                for s in range(2):
                    def body(l, carry, s=s, buf=buf):
                        r = s * L + l
                        return tuple(carry[c] + buf[r, pl.ds(c * LANES, LANES)]
                                     for c in range(nch))
                    acc = lax.fori_loop(
                        0, L, body,
                        tuple(jnp.zeros((LANES,), jnp.float32)
                              for _ in range(nch)))
                    for c in range(nch):
                        acc_v[2 * jj + s, pl.ds(c * LANES, LANES)] = acc[c]
                nxt = j + b + 2

                @pl.when(nxt < PAIRS)
                def _(emb=emb, buf=buf, sem=sem, nxt=nxt):
                    pltpu.async_copy(
                        emb.at[idx_v.at[nxt, pl.ds(0, ROWS)]], buf, sem)

        pltpu.sync_copy(acc_v, out_hbm.at[t].at[pl.ds(wid * SPW, SPW)])


def _mlp_body(p_ref, w1_ref, b1_ref, w2_ref, b2_ref, o_ref):
    p = p_ref[...]
    w1 = w1_ref[...]
    h = (jnp.dot(p[0], w1[0], preferred_element_type=jnp.float32)
         + jnp.dot(p[1], w1[1], preferred_element_type=jnp.float32)
         + jnp.dot(p[2], w1[2], preferred_element_type=jnp.float32)
         + b1_ref[...])
    h = jnp.maximum(h, 0.0)
    o_ref[...] = (jnp.dot(h, w2_ref[...], preferred_element_type=jnp.float32)
                  + b2_ref[...])


def kernel(x, emb1, emb2, emb3, W1, b1, W2, b2):
    x = x.astype(jnp.int32)
    idx = x.reshape(3, B // 2, ROWS)
    idx = jnp.pad(idx, ((0, 0), (0, 0), (0, IDX_PAD - ROWS)))
    idx = idx.reshape(3, NW, PAIRS, IDX_PAD)

    mesh = plsc.VectorSubcoreMesh(core_axis_name="c", subcore_axis_name="s")
    pooled = pl.kernel(
        _pool_body,
        out_type=jax.ShapeDtypeStruct((3, B, D), jnp.float32),
        mesh=mesh,
        scratch_types=[
            pltpu.VMEM((PAIRS, IDX_PAD), jnp.int32),
            pltpu.VMEM((ROWS, D), jnp.float32),
            pltpu.VMEM((ROWS, D), jnp.float32),
            pltpu.VMEM((SPW, D), jnp.float32),
            pltpu.SemaphoreType.DMA,
            pltpu.SemaphoreType.DMA,
        ],
    )(emb1, emb2, emb3, idx)

    w1s = (W1 * (1.0 / L)).reshape(3, D, H)
    b1r = b1.reshape(1, H)
    b2r = b2.reshape(1, C)

    out = pl.pallas_call(
        _mlp_body,
        grid=(B // BT,),
        in_specs=[
            pl.BlockSpec((3, BT, D), lambda i: (0, i, 0)),
            pl.BlockSpec((3, D, H), lambda i: (0, 0, 0)),
            pl.BlockSpec((1, H), lambda i: (0, 0)),
            pl.BlockSpec((H, C), lambda i: (0, 0)),
            pl.BlockSpec((1, C), lambda i: (0, 0)),
        ],
        out_specs=pl.BlockSpec((BT, C), lambda i: (i, 0)),
        out_shape=jax.ShapeDtypeStruct((B, C), jnp.float32),
    )(pooled, w1s, b1r, W2, b2r)
    return out


# speedup vs baseline: 11.6258x; 1.0028x over previous
import functools

import jax
import jax.numpy as jnp
from jax import lax
from jax.experimental import pallas as pl
from jax.experimental.pallas import tpu as pltpu
from jax.experimental.pallas import tpu_sc as plsc

VOCAB = 100000
D = 128
H = 32
C = 1000
B = 4096
L = 50

NC = 2
NS = 16
LANES = 16
NW = NC * NS
SPW = B // NW
PAIRS = SPW // 2
ROWS = 2 * L
BT = 256


def _pool_body(e1_hbm, e2_hbm, e3_hbm, idx_hbm, out_hbm,
               idx_v, rows0, rows1, acc_v, sem0, sem1):
    wid = lax.axis_index("s") * NC + lax.axis_index("c")
    nch = D // LANES
    bufs = ((rows0, sem0), (rows1, sem1))

    for t, emb in enumerate((e1_hbm, e2_hbm, e3_hbm)):
        pltpu.sync_copy(idx_hbm.at[t, wid], idx_v)

        for b, (buf, sem) in enumerate(bufs):
            pltpu.async_copy(emb.at[idx_v.at[b]], buf, sem)

        @pl.loop(0, PAIRS, step=2)
        def _(j, emb=emb):
            for b, (buf, sem) in enumerate(bufs):
                jj = j + b
                pltpu.make_async_copy(
                    emb.at[idx_v.at[0]], buf, sem).wait()
                for s in range(2):
                    def body(l, carry, s=s, buf=buf):
                        r = s * L + l
                        return tuple(carry[c] + buf[r, pl.ds(c * LANES, LANES)]
                                     for c in range(nch))
                    acc = lax.fori_loop(
                        0, L, body,
                        tuple(jnp.zeros((LANES,), jnp.float32)
                              for _ in range(nch)))
                    for c in range(nch):
                        acc_v[2 * jj + s, pl.ds(c * LANES, LANES)] = acc[c]
                nxt = j + b + 2

                @pl.when(nxt < PAIRS)
                def _(emb=emb, buf=buf, sem=sem, nxt=nxt):
                    pltpu.async_copy(emb.at[idx_v.at[nxt]], buf, sem)

        pltpu.sync_copy(acc_v, out_hbm.at[t].at[pl.ds(wid * SPW, SPW)])


def _mlp_body(p_ref, w1_ref, b1_ref, w2_ref, b2_ref, o_ref):
    p = p_ref[...]
    w1 = w1_ref[...]
    h = (jnp.dot(p[0], w1[0:D], preferred_element_type=jnp.float32)
         + jnp.dot(p[1], w1[D:2 * D], preferred_element_type=jnp.float32)
         + jnp.dot(p[2], w1[2 * D:], preferred_element_type=jnp.float32))
    h = jnp.maximum(h * (1.0 / L) + b1_ref[...], 0.0)
    o_ref[...] = (jnp.dot(h, w2_ref[...], preferred_element_type=jnp.float32)
                  + b2_ref[...])


def kernel(x, emb1, emb2, emb3, W1, b1, W2, b2):
    x = x.astype(jnp.int32)
    idx = x.reshape(3, NW, PAIRS, ROWS)

    mesh = plsc.VectorSubcoreMesh(core_axis_name="c", subcore_axis_name="s")
    pooled = pl.kernel(
        _pool_body,
        out_type=jax.ShapeDtypeStruct((3, B, D), jnp.float32),
        mesh=mesh,
        scratch_types=[
            pltpu.VMEM((PAIRS, ROWS), jnp.int32),
            pltpu.VMEM((ROWS, D), jnp.float32),
            pltpu.VMEM((ROWS, D), jnp.float32),
            pltpu.VMEM((SPW, D), jnp.float32),
            pltpu.SemaphoreType.DMA,
            pltpu.SemaphoreType.DMA,
        ],
    )(emb1, emb2, emb3, idx)

    b1r = b1.reshape(1, H)
    b2r = b2.reshape(1, C)

    out = pl.pallas_call(
        _mlp_body,
        grid=(B // BT,),
        in_specs=[
            pl.BlockSpec((3, BT, D), lambda i: (0, i, 0)),
            pl.BlockSpec((3 * D, H), lambda i: (0, 0)),
            pl.BlockSpec((1, H), lambda i: (0, 0)),
            pl.BlockSpec((H, C), lambda i: (0, 0)),
            pl.BlockSpec((1, C), lambda i: (0, 0)),
        ],
        out_specs=pl.BlockSpec((BT, C), lambda i: (i, 0)),
        out_shape=jax.ShapeDtypeStruct((B, C), jnp.float32),
    )(pooled, W1, b1r, W2, b2r)
    return out
